# Initial kernel scaffold; baseline (speedup 1.0000x reference)
import jax
import jax.numpy as jnp
from jax.experimental import pallas as pl


def kernel(x, edge_index, batch, lmax, W1, b1, g1, be1, rm1, rv1, W2, b2, g2, be2, rm2, rv2, Wfc, bfc):
    raise NotImplementedError("write your pallas kernel here")


# speedup vs baseline: 4.9502x; 4.9502x over previous
import functools

import jax
import jax.numpy as jnp
from jax import lax
from jax.experimental import pallas as pl
from jax.experimental.pallas import tpu as pltpu
from jax.experimental.pallas import tpu_sc as plsc

N = 10000
E = 320000
F_IN = 128
H = 200
HP = 208
G = 64
EPS = 1e-5

NC = 2
NS = 16
NW = NC * NS
GRP = 79
EPT = GRP * 128
EP = NW * EPT
NDEG = 10112
DSTR = NDEG // NS
NAGG = 10016
BSTR = 632

_MESH = dict(core_axis_name="c", subcore_axis_name="s")


def _sc_mesh():
    return plsc.VectorSubcoreMesh(**_MESH)


def _hist(rowp4):
    @functools.partial(
        pl.kernel,
        out_type=jax.ShapeDtypeStruct((NC * NDEG,), jnp.float32),
        mesh=_sc_mesh(),
        compiler_params=pltpu.CompilerParams(needs_layout_passes=False, use_tc_tiling_on_sc=False),
        scratch_types=[
            pltpu.VMEM((GRP, 1, 128), jnp.int32),
            pltpu.VMEM((128,), jnp.float32),
            pltpu.VMEM((640,), jnp.float32),
            pltpu.VMEM_SHARED((NDEG,), jnp.float32),
        ],
    )
    def k(rowp_hbm, out_hbm, row_v, ones_v, z_v, deg_sh):
        c = lax.axis_index("c")
        s = lax.axis_index("s")
        wid = c * NS + s
        for j in range(8):
            ones_v[pl.ds(16 * j, 16)] = jnp.ones((16,), jnp.float32)
        for j in range(40):
            z_v[pl.ds(16 * j, 16)] = jnp.zeros((16,), jnp.float32)
        pltpu.sync_copy(rowp_hbm.at[wid], row_v)
        pltpu.sync_copy(z_v.at[pl.ds(0, DSTR)], deg_sh.at[pl.ds(s * DSTR, DSTR)])
        plsc.subcore_barrier()

        def body(g, _):
            pltpu.sync_copy(ones_v, deg_sh.at[row_v.at[g, 0]], add=True)
            return 0

        lax.fori_loop(0, GRP, body, 0)
        plsc.subcore_barrier()
        pltpu.sync_copy(deg_sh.at[pl.ds(s * DSTR, DSTR)],
                        z_v.at[pl.ds(0, DSTR)])
        pltpu.sync_copy(z_v.at[pl.ds(0, DSTR)],
                        out_hbm.at[pl.ds(c * NDEG + s * DSTR, DSTR)])

    return k(rowp4)


def _edge_w(dis, diag, rowf, colf):
    @functools.partial(
        pl.kernel,
        out_type=jax.ShapeDtypeStruct((EP,), jnp.float32),
        mesh=_sc_mesh(),
        compiler_params=pltpu.CompilerParams(needs_layout_passes=False, use_tc_tiling_on_sc=False),
        scratch_types=[
            pltpu.VMEM((NDEG,), jnp.float32),
            pltpu.VMEM((NDEG,), jnp.float32),
            pltpu.VMEM((EPT,), jnp.int32),
            pltpu.VMEM((EPT,), jnp.int32),
            pltpu.VMEM((EPT,), jnp.float32),
        ],
    )
    def k(dis_hbm, diag_hbm, row_hbm, col_hbm, w_hbm,
          dis_v, diag_v, row_v, col_v, w_v):
        c = lax.axis_index("c")
        s = lax.axis_index("s")
        wid = c * NS + s
        pltpu.sync_copy(dis_hbm, dis_v)
        pltpu.sync_copy(diag_hbm, diag_v)
        pltpu.sync_copy(row_hbm.at[pl.ds(wid * EPT, EPT)], row_v)
        pltpu.sync_copy(col_hbm.at[pl.ds(wid * EPT, EPT)], col_v)

        def body(g, _):
            for j in range(8):
                off = g * 128 + 16 * j
                r16 = row_v[pl.ds(off, 16)]
                c16 = col_v[pl.ds(off, 16)]
                dr = plsc.load_gather(dis_v, [r16])
                dc = plsc.load_gather(dis_v, [c16])
                gr = plsc.load_gather(diag_v, [r16])
                w_v[pl.ds(off, 16)] = -(dr * dc) * (gr + 1.0)
            return 0

        lax.fori_loop(0, GRP, body, 0)
        pltpu.sync_copy(w_v, w_hbm.at[pl.ds(wid * EPT, EPT)])

    return k(dis, diag, rowf, colf)


def _propagate(v, colp4, rowp4, wf, F):
    @functools.partial(
        pl.kernel,
        out_type=jax.ShapeDtypeStruct((NC, NAGG, F), jnp.float32),
        mesh=_sc_mesh(),
        compiler_params=pltpu.CompilerParams(needs_layout_passes=False, use_tc_tiling_on_sc=False),
        scratch_types=[
            pltpu.VMEM((GRP, 1, 128), jnp.int32),
            pltpu.VMEM((GRP, 1, 128), jnp.int32),
            pltpu.VMEM((EPT,), jnp.float32),
            pltpu.VMEM((128, F), jnp.float32),
            pltpu.VMEM_SHARED((NAGG, F), jnp.float32),
            pltpu.SemaphoreType.DMA,
        ],
    )
    def k(v_hbm, col_hbm, row_hbm, w_hbm, out_hbm,
          col_v, row_v, w_v, gbuf, agg_sh, gsem):
        c = lax.axis_index("c")
        s = lax.axis_index("s")
        wid = c * NS + s
        pltpu.sync_copy(col_hbm.at[wid], col_v)
        pltpu.sync_copy(row_hbm.at[wid], row_v)
        pltpu.sync_copy(w_hbm.at[pl.ds(wid * EPT, EPT)], w_v)

        def zrow(i, _):
            for j in range(F // 16):
                gbuf[i, pl.ds(16 * j, 16)] = jnp.zeros((16,), jnp.float32)
            return 0

        lax.fori_loop(0, 128, zrow, 0)
        base = s * BSTR
        for kk in range(4):
            pltpu.sync_copy(gbuf.at[pl.ds(0, 128)],
                            agg_sh.at[pl.ds(base + 128 * kk, 128)])
        pl.when(s < NS - 1)(lambda: pltpu.sync_copy(
            gbuf.at[pl.ds(0, BSTR - 512)],
            agg_sh.at[pl.ds(base + 512, BSTR - 512)]))
        pl.when(s == NS - 1)(lambda: pltpu.sync_copy(
            gbuf.at[pl.ds(0, 24)],
            agg_sh.at[pl.ds(base + 512, 24)]))
        plsc.subcore_barrier()

        def body(g, _):
            pltpu.async_copy(v_hbm.at[col_v.at[g, 0]], gbuf, gsem).wait()

            def scale(r16, _2):
                for rr in range(16):
                    r = r16 * 16 + rr
                    wb = plsc.load_gather(
                        w_v, [jnp.full((16,), g * 128 + r, jnp.int32)])
                    for j in range(F // 16):
                        gbuf[r, pl.ds(16 * j, 16)] = (
                            gbuf[r, pl.ds(16 * j, 16)] * wb)
                return 0

            lax.fori_loop(0, 8, scale, 0)
            pltpu.sync_copy(gbuf, agg_sh.at[row_v.at[g, 0]], add=True)
            return 0

        lax.fori_loop(0, GRP, body, 0)
        plsc.subcore_barrier()

        def out_rows(n0, cnt):
            pltpu.sync_copy(agg_sh.at[pl.ds(n0, cnt)], gbuf.at[pl.ds(0, cnt)])
            pltpu.sync_copy(gbuf.at[pl.ds(0, cnt)],
                            out_hbm.at[c, pl.ds(n0, cnt)])

        for kk in range(4):
            out_rows(base + 128 * kk, 128)
        pl.when(s < NS - 1)(lambda: out_rows(base + 512, BSTR - 512))
        pl.when(s == NS - 1)(lambda: out_rows(base + 512, 24))

    return k(v, colp4, rowp4, wf)


def _prep(degp, batch2d, lmax2d):
    def body(degp_ref, b_ref, lm_ref, dis_ref, diag_ref):
        deg = degp_ref[:GRP, :] + degp_ref[GRP:, :]
        idx = (lax.broadcasted_iota(jnp.int32, (GRP, 128), 0) * 128
               + lax.broadcasted_iota(jnp.int32, (GRP, 128), 1))
        valid = idx < N
        dis = jnp.where((deg > 0) & valid,
                        lax.rsqrt(jnp.maximum(deg, 1.0)), 0.0)
        dis_ref[...] = dis
        b = b_ref[...]
        lm = lm_ref[...]
        lb = jnp.zeros((GRP, 128), jnp.float32)
        for g in range(G):
            lb = jnp.where(b == g, lm[0:1, g:g + 1], lb)
        diag_ref[...] = jnp.where(valid, 2.0 / jnp.maximum(lb, 1e-6) - 1.0, 0.0)

    return pl.pallas_call(
        body,
        out_shape=[jax.ShapeDtypeStruct((GRP, 128), jnp.float32),
                   jax.ShapeDtypeStruct((GRP, 128), jnp.float32)],
    )(degp, batch2d, lmax2d)


def _combine(s0, s1, v, diagc, F):
    def body(a_ref, b_ref, v_ref, d_ref, o_ref):
        o_ref[...] = a_ref[...] + b_ref[...] + d_ref[...] * v_ref[...]

    blk = 1000
    return pl.pallas_call(
        body,
        grid=(N // blk,),
        in_specs=[pl.BlockSpec((blk, F), lambda i: (i, 0)),
                  pl.BlockSpec((blk, F), lambda i: (i, 0)),
                  pl.BlockSpec((blk, F), lambda i: (i, 0)),
                  pl.BlockSpec((blk, 1), lambda i: (i, 0))],
        out_specs=pl.BlockSpec((blk, F), lambda i: (i, 0)),
        out_shape=jax.ShapeDtypeStruct((N, F), jnp.float32),
    )(s0, s1, v, diagc)


def _layer_out(tx0, tx1, s0, s1, diagc, W, b, ga, be, rm, rv, Fin):
    def body(t0_ref, t1_ref, a_ref, bb_ref, d_ref, w_ref, bias_ref,
             g_ref, be_ref, rm_ref, rv_ref, o_ref):
        t0 = t0_ref[...]
        t1 = t1_ref[...]
        t2 = 2.0 * (a_ref[...] + bb_ref[...] + d_ref[...] * t1) - t0
        acc = (jnp.dot(t0, w_ref[0], preferred_element_type=jnp.float32)
               + jnp.dot(t1, w_ref[1], preferred_element_type=jnp.float32)
               + jnp.dot(t2, w_ref[2], preferred_element_type=jnp.float32)
               + bias_ref[...])
        r = jnp.maximum(acc, 0.0)
        sc = g_ref[...] * lax.rsqrt(rv_ref[...] + EPS)
        h = r * sc + (be_ref[...] - rm_ref[...] * sc)
        o_ref[...] = jnp.concatenate(
            [h, jnp.zeros((h.shape[0], HP - H), jnp.float32)], axis=1)

    blk = 1000
    return pl.pallas_call(
        body,
        grid=(N // blk,),
        in_specs=[pl.BlockSpec((blk, Fin), lambda i: (i, 0)),
                  pl.BlockSpec((blk, Fin), lambda i: (i, 0)),
                  pl.BlockSpec((blk, Fin), lambda i: (i, 0)),
                  pl.BlockSpec((blk, Fin), lambda i: (i, 0)),
                  pl.BlockSpec((blk, 1), lambda i: (i, 0)),
                  pl.BlockSpec((3, Fin, H), lambda i: (0, 0, 0)),
                  pl.BlockSpec((1, H), lambda i: (0, 0)),
                  pl.BlockSpec((1, H), lambda i: (0, 0)),
                  pl.BlockSpec((1, H), lambda i: (0, 0)),
                  pl.BlockSpec((1, H), lambda i: (0, 0)),
                  pl.BlockSpec((1, H), lambda i: (0, 0))],
        out_specs=pl.BlockSpec((blk, HP), lambda i: (i, 0)),
        out_shape=jax.ShapeDtypeStruct((N, HP), jnp.float32),
    )(tx0, tx1, s0, s1, diagc, W, b, ga, be, rm, rv)


def _pool(h, batchr, batchc, Wfc, bfc):
    def body(h_ref, br_ref, bc_ref, w_ref, bias_ref, o_ref):
        hh = h_ref[...][:, :H]
        br = br_ref[...]
        bc = bc_ref[...]
        onehot_t = (lax.broadcasted_iota(jnp.int32, (G, N), 0)
                    == jnp.broadcast_to(br, (G, N))).astype(jnp.float32)
        sums = jnp.dot(onehot_t, hh, preferred_element_type=jnp.float32)
        cnt = jnp.dot(onehot_t, jnp.ones((N, 1), jnp.float32),
                      preferred_element_type=jnp.float32)
        mean = sums / jnp.maximum(cnt, 1.0)
        rows = []
        for g in range(G):
            m = jnp.where(bc == g, hh, -3.4e38)
            rows.append(jnp.max(m, axis=0, keepdims=True))
        mx = jnp.concatenate(rows, axis=0)
        mx = jnp.where(cnt > 0, mx, 0.0)
        logits = (jnp.dot(mean, w_ref[...][:H, :],
                          preferred_element_type=jnp.float32)
                  + jnp.dot(mx, w_ref[...][H:, :],
                            preferred_element_type=jnp.float32)
                  + bias_ref[...])
        mmax = jnp.max(logits, axis=1, keepdims=True)
        lse = mmax + jnp.log(jnp.sum(jnp.exp(logits - mmax),
                                     axis=1, keepdims=True))
        o_ref[...] = logits - lse

    return pl.pallas_call(
        body,
        out_shape=jax.ShapeDtypeStruct((G, 2), jnp.float32),
    )(h, batchr, batchc, Wfc, bfc)


def kernel(x, edge_index, batch, lmax, W1, b1, g1, be1, rm1, rv1,
           W2, b2, g2, be2, rm2, rv2, Wfc, bfc):
    row = edge_index[0]
    col = edge_index[1]
    rowp = jnp.pad(row, (0, EP - E), constant_values=N)
    colp = jnp.pad(col, (0, EP - E), constant_values=0)
    rowp4 = rowp.reshape(NW, GRP, 1, 128)
    colp4 = colp.reshape(NW, GRP, 1, 128)

    degp = _hist(rowp4)
    batch2d = jnp.pad(batch, (0, NDEG - N)).reshape(GRP, 128)
    dis2d, diag2d = _prep(degp.reshape(2 * GRP, 128), batch2d,
                          lmax.reshape(1, G))
    dis = dis2d.reshape(NDEG)
    diag = diag2d.reshape(NDEG)
    diagc = diag[:N].reshape(N, 1)

    wf = _edge_w(dis, diag, rowp, colp)

    def prop_parts(v, Fin):
        if Fin <= 128:
            sp = _propagate(v, colp4, rowp4, wf, Fin)
            return sp[0, :N], sp[1, :N]
        spl = _propagate(v[:, :128], colp4, rowp4, wf, 128)
        sph = _propagate(v[:, 128:], colp4, rowp4, wf, Fin - 128)
        s0 = jnp.concatenate([spl[0, :N], sph[0, :N]], axis=1)
        s1 = jnp.concatenate([spl[1, :N], sph[1, :N]], axis=1)
        return s0, s1

    def cheb_layer(v, W, b, ga, bee, rmm, rvv, Fin):
        s0, s1 = prop_parts(v, Fin)
        tx1 = _combine(s0, s1, v, diagc, Fin)
        t0, t1 = prop_parts(tx1, Fin)
        return _layer_out(v, tx1, t0, t1, diagc,
                          W, b.reshape(1, H), ga.reshape(1, H),
                          bee.reshape(1, H), rmm.reshape(1, H),
                          rvv.reshape(1, H), Fin)

    h1 = cheb_layer(x, W1, b1, g1, be1, rm1, rv1, F_IN)
    W2p = jnp.pad(W2, ((0, 0), (0, HP - H), (0, 0)))
    h2 = cheb_layer(h1, W2p, b2, g2, be2, rm2, rv2, HP)

    return _pool(h2, batch.reshape(1, N), batch.reshape(N, 1),
                 Wfc, bfc.reshape(1, 2))
